# Initial kernel scaffold; baseline (speedup 1.0000x reference)
import jax
import jax.numpy as jnp
from jax.experimental import pallas as pl


def kernel(embeddings, input_mask, conversation_mask, options_tensor, gold_next, gold_prev, w_proj, b_proj, w_next, b_next, w_prev, b_prev):
    raise NotImplementedError("write your pallas kernel here")


# speedup vs baseline: 5.9011x; 5.9011x over previous
import jax
import jax.numpy as jnp
from jax import lax
from jax.experimental import pallas as pl
from jax.experimental.pallas import tpu as pltpu

_VMEM_LIMIT = 64 * 1024 * 1024


def _pick_tile(n, target):
    t = min(target, n)
    while n % t:
        t -= 1
    return t


def _encode_kernel(emb_ref, mask_ref, wp_ref, bp_ref, wcat_ref, enc_ref, v_ref):
    mask = mask_ref[...]
    denom = jnp.maximum(jnp.sum(mask, axis=1, keepdims=True), 1.0)
    pooled = jnp.sum(emb_ref[...] * mask[:, :, None], axis=1) / denom
    enc = (jnp.dot(pooled, wp_ref[...], preferred_element_type=jnp.float32)
           + bp_ref[...])
    enc_ref[...] = enc
    v_ref[...] = jnp.dot(enc, wcat_ref[...],
                         preferred_element_type=jnp.float32)


def _score_kernel(v_ref, enc_ref, opt_ref, gn_ref, gp_ref, cm_ref, bias_ref,
                  nextp_ref, prevp_ref, num_ref, den_ref):
    v = v_ref[...]
    enc = enc_ref[...]
    tN = v.shape[0]
    N, H = enc.shape
    opts = opt_ref[...]
    O = opts.shape[1]

    dims = (((1,), (1,)), ((), ()))
    s_next = lax.dot_general(v[:, :H], enc, dims,
                             preferred_element_type=jnp.float32)
    s_prev = lax.dot_general(v[:, H:], enc, dims,
                             preferred_element_type=jnp.float32)

    col = lax.broadcasted_iota(jnp.int32, (tN, N), 1)
    zero = jnp.zeros_like(s_next)
    nl, pv = [], []
    for o in range(O):
        hit = col == opts[:, o:o + 1]
        nl.append(jnp.sum(jnp.where(hit, s_next, zero), axis=1, keepdims=True))
        pv.append(jnp.sum(jnp.where(hit, s_prev, zero), axis=1, keepdims=True))
    next_logits = jnp.concatenate(nl, axis=1) + bias_ref[0]
    prev_logits = jnp.concatenate(pv, axis=1) + bias_ref[1]

    def log_softmax(x):
        m = jnp.max(x, axis=-1, keepdims=True)
        s = x - m
        return s - jnp.log(jnp.sum(jnp.exp(s), axis=-1, keepdims=True))

    next_lp = log_softmax(next_logits)
    prev_lp = log_softmax(prev_logits)

    col_o = lax.broadcasted_iota(jnp.int32, (tN, O), 1)
    oh_gn = (col_o == gn_ref[...]).astype(jnp.float32)
    oh_gp = (col_o == gp_ref[...]).astype(jnp.float32)
    losses = (-jnp.sum(next_lp * oh_gn, axis=-1, keepdims=True)
              - jnp.sum(prev_lp * oh_gp, axis=-1, keepdims=True))

    cm = cm_ref[...]
    num_ref[...] = jnp.sum(losses * cm).reshape(1, 1, 1)
    den_ref[...] = jnp.sum(cm).reshape(1, 1, 1)

    def argmax_first(x):
        m = jnp.max(x, axis=-1, keepdims=True)
        idx = jnp.where(x == m, col_o, jnp.int32(O))
        return jnp.min(idx, axis=-1, keepdims=True)

    nextp_ref[...] = argmax_first(next_logits)
    prevp_ref[...] = argmax_first(prev_logits)


def kernel(embeddings, input_mask, conversation_mask, options_tensor,
           gold_next, gold_prev, w_proj, b_proj, w_next, b_next,
           w_prev, b_prev):
    N, T, E = embeddings.shape
    H = w_proj.shape[1]
    O = options_tensor.shape[1]

    w_cat = jnp.concatenate([w_next, w_prev], axis=1)

    te = _pick_tile(N, 256)
    enc, v = pl.pallas_call(
        _encode_kernel,
        out_shape=(jax.ShapeDtypeStruct((N, H), jnp.float32),
                   jax.ShapeDtypeStruct((N, 2 * H), jnp.float32)),
        grid=(N // te,),
        in_specs=[
            pl.BlockSpec((te, T, E), lambda i: (i, 0, 0)),
            pl.BlockSpec((te, T), lambda i: (i, 0)),
            pl.BlockSpec((E, H), lambda i: (0, 0)),
            pl.BlockSpec((1, H), lambda i: (0, 0)),
            pl.BlockSpec((H, 2 * H), lambda i: (0, 0)),
        ],
        out_specs=(pl.BlockSpec((te, H), lambda i: (i, 0)),
                   pl.BlockSpec((te, 2 * H), lambda i: (i, 0))),
        compiler_params=pltpu.CompilerParams(
            dimension_semantics=("parallel",),
            vmem_limit_bytes=_VMEM_LIMIT),
    )(embeddings, input_mask, w_proj, b_proj.reshape(1, H), w_cat)

    ts = _pick_tile(N, 256)
    G = N // ts
    biases = jnp.concatenate([b_next, b_prev]).astype(jnp.float32)
    cm = conversation_mask.reshape(N, 1).astype(jnp.float32)
    gn = gold_next.reshape(N, 1).astype(jnp.int32)
    gp = gold_prev.reshape(N, 1).astype(jnp.int32)

    nextp, prevp, num, den = pl.pallas_call(
        _score_kernel,
        out_shape=(jax.ShapeDtypeStruct((N, 1), jnp.int32),
                   jax.ShapeDtypeStruct((N, 1), jnp.int32),
                   jax.ShapeDtypeStruct((G, 1, 1), jnp.float32),
                   jax.ShapeDtypeStruct((G, 1, 1), jnp.float32)),
        grid=(G,),
        in_specs=[
            pl.BlockSpec((ts, 2 * H), lambda i: (i, 0)),
            pl.BlockSpec((N, H), lambda i: (0, 0)),
            pl.BlockSpec((ts, O), lambda i: (i, 0)),
            pl.BlockSpec((ts, 1), lambda i: (i, 0)),
            pl.BlockSpec((ts, 1), lambda i: (i, 0)),
            pl.BlockSpec((ts, 1), lambda i: (i, 0)),
            pl.BlockSpec(memory_space=pltpu.MemorySpace.SMEM),
        ],
        out_specs=(pl.BlockSpec((ts, 1), lambda i: (i, 0)),
                   pl.BlockSpec((ts, 1), lambda i: (i, 0)),
                   pl.BlockSpec((1, 1, 1), lambda i: (i, 0, 0)),
                   pl.BlockSpec((1, 1, 1), lambda i: (i, 0, 0))),
        compiler_params=pltpu.CompilerParams(
            dimension_semantics=("parallel",),
            vmem_limit_bytes=_VMEM_LIMIT),
    )(v, enc, options_tensor.astype(jnp.int32), gn, gp, cm, biases)

    loss = jnp.sum(num) / (2.0 * jnp.sum(den))
    return loss, (nextp[:, 0], prevp[:, 0])


# speedup vs baseline: 6.5197x; 1.1048x over previous
import jax
import jax.numpy as jnp
from jax import lax
from jax.experimental import pallas as pl
from jax.experimental.pallas import tpu as pltpu

_VMEM_LIMIT = 64 * 1024 * 1024


def _pick_tile(n, target):
    t = min(target, n)
    while n % t:
        t -= 1
    return t


def _fused_kernel(emb_ref, mask_ref, wp_ref, bp_ref, wcat_ref,
                  opt_ref, gn_ref, gp_ref, cm_ref, bias_ref,
                  loss_ref, nextp_ref, prevp_ref,
                  enc_all, v_all, num_acc, den_acc):
    step = pl.program_id(0)
    G = pl.num_programs(0) // 2
    tN = nextp_ref.shape[0]
    N, H = enc_all.shape

    @pl.when(step == 0)
    def _():
        num_acc[...] = jnp.zeros_like(num_acc)
        den_acc[...] = jnp.zeros_like(den_acc)

    @pl.when(step < G)
    def _encode():
        mask = mask_ref[...]
        denom = jnp.maximum(jnp.sum(mask, axis=1, keepdims=True), 1.0)
        pooled = jnp.sum(emb_ref[...] * mask[:, :, None], axis=1) / denom
        enc = (jnp.dot(pooled, wp_ref[...], preferred_element_type=jnp.float32)
               + bp_ref[...])
        row = pl.multiple_of(step * tN, 8)
        enc_all[pl.ds(row, tN), :] = enc
        v_all[pl.ds(row, tN), :] = jnp.dot(
            enc, wcat_ref[...], preferred_element_type=jnp.float32)

    @pl.when(step >= G)
    def _score():
        row = pl.multiple_of((step - G) * tN, 8)
        v = v_all[pl.ds(row, tN), :]
        enc = enc_all[...]
        opts = opt_ref[...]
        O = opts.shape[1]

        dims = (((1,), (1,)), ((), ()))
        s_next = lax.dot_general(v[:, :H], enc, dims,
                                 preferred_element_type=jnp.float32)
        s_prev = lax.dot_general(v[:, H:], enc, dims,
                                 preferred_element_type=jnp.float32)

        col = lax.broadcasted_iota(jnp.int32, (tN, N), 1)
        zero = jnp.zeros_like(s_next)
        nl, pv = [], []
        for o in range(O):
            hit = col == opts[:, o:o + 1]
            nl.append(jnp.sum(jnp.where(hit, s_next, zero),
                              axis=1, keepdims=True))
            pv.append(jnp.sum(jnp.where(hit, s_prev, zero),
                              axis=1, keepdims=True))
        next_logits = jnp.concatenate(nl, axis=1) + bias_ref[0]
        prev_logits = jnp.concatenate(pv, axis=1) + bias_ref[1]

        def log_softmax(x):
            m = jnp.max(x, axis=-1, keepdims=True)
            s = x - m
            return s - jnp.log(jnp.sum(jnp.exp(s), axis=-1, keepdims=True))

        next_lp = log_softmax(next_logits)
        prev_lp = log_softmax(prev_logits)

        col_o = lax.broadcasted_iota(jnp.int32, (tN, O), 1)
        oh_gn = (col_o == gn_ref[...]).astype(jnp.float32)
        oh_gp = (col_o == gp_ref[...]).astype(jnp.float32)
        losses = (-jnp.sum(next_lp * oh_gn, axis=-1, keepdims=True)
                  - jnp.sum(prev_lp * oh_gp, axis=-1, keepdims=True))

        cm = cm_ref[...]
        num_acc[...] += jnp.sum(losses * cm).reshape(1, 1)
        den_acc[...] += jnp.sum(cm).reshape(1, 1)

        def argmax_first(x):
            m = jnp.max(x, axis=-1, keepdims=True)
            idx = jnp.where(x == m, col_o, jnp.int32(O))
            return jnp.min(idx, axis=-1, keepdims=True)

        nextp_ref[...] = argmax_first(next_logits)
        prevp_ref[...] = argmax_first(prev_logits)

    @pl.when(step == pl.num_programs(0) - 1)
    def _():
        loss_ref[...] = num_acc[...] / (2.0 * den_acc[...])


def kernel(embeddings, input_mask, conversation_mask, options_tensor,
           gold_next, gold_prev, w_proj, b_proj, w_next, b_next,
           w_prev, b_prev):
    N, T, E = embeddings.shape
    H = w_proj.shape[1]
    O = options_tensor.shape[1]

    w_cat = jnp.concatenate([w_next, w_prev], axis=1)
    biases = jnp.concatenate([b_next, b_prev]).astype(jnp.float32)
    cm = conversation_mask.reshape(N, 1).astype(jnp.float32)
    gn = gold_next.reshape(N, 1).astype(jnp.int32)
    gp = gold_prev.reshape(N, 1).astype(jnp.int32)

    tN = _pick_tile(N, 256)
    G = N // tN

    def enc_map(i):
        return (jnp.minimum(i, G - 1), 0, 0)

    def enc_map2(i):
        return (jnp.minimum(i, G - 1), 0)

    def score_map(i):
        return (jnp.maximum(i - G, 0), 0)

    loss, nextp, prevp = pl.pallas_call(
        _fused_kernel,
        out_shape=(jax.ShapeDtypeStruct((1, 1), jnp.float32),
                   jax.ShapeDtypeStruct((N, 1), jnp.int32),
                   jax.ShapeDtypeStruct((N, 1), jnp.int32)),
        grid=(2 * G,),
        in_specs=[
            pl.BlockSpec((tN, T, E), enc_map),
            pl.BlockSpec((tN, T), enc_map2),
            pl.BlockSpec((E, H), lambda i: (0, 0)),
            pl.BlockSpec((1, H), lambda i: (0, 0)),
            pl.BlockSpec((H, 2 * H), lambda i: (0, 0)),
            pl.BlockSpec((tN, O), score_map),
            pl.BlockSpec((tN, 1), score_map),
            pl.BlockSpec((tN, 1), score_map),
            pl.BlockSpec((tN, 1), score_map),
            pl.BlockSpec(memory_space=pltpu.MemorySpace.SMEM),
        ],
        out_specs=(pl.BlockSpec((1, 1), lambda i: (0, 0)),
                   pl.BlockSpec((tN, 1), score_map),
                   pl.BlockSpec((tN, 1), score_map)),
        scratch_shapes=[pltpu.VMEM((N, H), jnp.float32),
                        pltpu.VMEM((N, 2 * H), jnp.float32),
                        pltpu.VMEM((1, 1), jnp.float32),
                        pltpu.VMEM((1, 1), jnp.float32)],
        compiler_params=pltpu.CompilerParams(
            dimension_semantics=("arbitrary",),
            vmem_limit_bytes=_VMEM_LIMIT),
    )(embeddings, input_mask, w_proj, b_proj.reshape(1, H), w_cat,
      options_tensor.astype(jnp.int32), gn, gp, cm, biases)

    return loss[0, 0], (nextp[:, 0], prevp[:, 0])
